# baseline (speedup 1.0000x reference)
import jax
import jax.numpy as jnp
from jax.experimental import pallas as pl
from jax.experimental.pallas import tpu as pltpu

_B = 128
_D = 64
_VR = 72
_BLOCK = 16384
_NBUF = 3


def _flash_kernel(x_ref, k_hbm, v_hbm, o_ref, *scratch):
    kb = scratch[0:_NBUF]
    vb = scratch[_NBUF:2 * _NBUF]
    m_s, l_s, acc_s = scratch[2 * _NBUF:2 * _NBUF + 3]
    ksem = scratch[2 * _NBUF + 3:3 * _NBUF + 3]
    vsem = scratch[3 * _NBUF + 3:4 * _NBUF + 3]

    n_blocks = k_hbm.shape[1] // _BLOCK

    def _start(j, slot):
        pltpu.make_async_copy(
            k_hbm.at[:, pl.ds(j * _BLOCK, _BLOCK)], kb[slot],
            ksem[slot]).start()
        pltpu.make_async_copy(
            v_hbm.at[:, pl.ds(j * _BLOCK, _BLOCK)], vb[slot].at[0:_D, :],
            vsem[slot]).start()

    def _wait(j, slot):
        pltpu.make_async_copy(
            k_hbm.at[:, pl.ds(j * _BLOCK, _BLOCK)], kb[slot],
            ksem[slot]).wait()
        pltpu.make_async_copy(
            v_hbm.at[:, pl.ds(j * _BLOCK, _BLOCK)], vb[slot].at[0:_D, :],
            vsem[slot]).wait()

    for slot in range(_NBUF):
        _start(slot, slot)
        vb[slot][_D:_VR, :] = jnp.ones((_VR - _D, _BLOCK), jnp.float32)

    m_s[...] = jnp.full_like(m_s, -jnp.inf)
    l_s[...] = jnp.zeros_like(l_s)
    acc_s[...] = jnp.zeros_like(acc_s)
    x = x_ref[...]

    for j in range(n_blocks):
        slot = j % _NBUF
        _wait(j, slot)

        k = kb[slot][...]
        vaug = vb[slot][...]

        s = jax.lax.dot_general(
            x, k, (((1,), (0,)), ((), ())),
            preferred_element_type=jnp.float32)

        m_prev = m_s[...]
        m_cur = jnp.max(s, axis=1, keepdims=True)
        m_new = jnp.maximum(m_prev, m_cur)

        alpha = jnp.exp(m_prev - m_new)
        p = jnp.exp(s - m_new[:, 0:1])

        pv_aug = jax.lax.dot_general(
            p, vaug, (((1,), (1,)), ((), ())),
            preferred_element_type=jnp.float32)

        l_s[...] = l_s[...] * alpha + pv_aug[:, _D:_D + 1]
        m_s[...] = m_new
        acc_s[...] = acc_s[...] * alpha[:, 0:1] + pv_aug[:, 0:_D]

        if j + _NBUF < n_blocks:
            _start(j + _NBUF, slot)

    o_ref[...] = acc_s[...] / l_s[...][:, 0:1]


def kernel(x, memory_keys, memory_values):
    kT = memory_keys.T
    vT = memory_values.T
    scratch = (
        [pltpu.VMEM((_D, _BLOCK), jnp.float32) for _ in range(_NBUF)]
        + [pltpu.VMEM((_VR, _BLOCK), jnp.float32) for _ in range(_NBUF)]
        + [pltpu.VMEM((_B, 128), jnp.float32),
           pltpu.VMEM((_B, 128), jnp.float32),
           pltpu.VMEM((_B, _D), jnp.float32)]
        + [pltpu.SemaphoreType.DMA for _ in range(2 * _NBUF)]
    )
    return pl.pallas_call(
        _flash_kernel,
        in_specs=[
            pl.BlockSpec(memory_space=pltpu.MemorySpace.VMEM),
            pl.BlockSpec(memory_space=pltpu.MemorySpace.HBM),
            pl.BlockSpec(memory_space=pltpu.MemorySpace.HBM),
        ],
        out_specs=pl.BlockSpec(memory_space=pltpu.MemorySpace.VMEM),
        out_shape=jax.ShapeDtypeStruct((_B, _D), jnp.float32),
        scratch_shapes=scratch,
    )(x, kT, vT)


# speedup vs baseline: 1.3092x; 1.3092x over previous
import jax
import jax.numpy as jnp
from jax.experimental import pallas as pl
from jax.experimental.pallas import tpu as pltpu

_B = 128
_D = 64
_BLOCK = 32768


def _flash_kernel(x_ref, k_ref, v_ref, o_ref, m_ref, l_ref, acc_ref):
    i = pl.program_id(0)
    n = pl.num_programs(0)

    @pl.when(i == 0)
    def _init():
        m_ref[...] = jnp.full_like(m_ref, -jnp.inf)
        l_ref[...] = jnp.zeros_like(l_ref)
        acc_ref[...] = jnp.zeros_like(acc_ref)

    x = x_ref[...].astype(jnp.bfloat16)
    kb = k_ref[...].astype(jnp.bfloat16)
    s = jax.lax.dot_general(
        x, kb, (((1,), (0,)), ((), ())),
        preferred_element_type=jnp.float32)

    m_prev = m_ref[...]
    m_cur = jnp.max(s, axis=1, keepdims=True)
    m_new = jnp.maximum(m_prev, m_cur)

    alpha = jnp.exp(m_prev - m_new)
    p = jnp.exp(s - m_new[:, 0:1])

    l_cur = jnp.sum(p, axis=1, keepdims=True)
    l_ref[...] = l_ref[...] * alpha + l_cur
    m_ref[...] = m_new

    pv = jax.lax.dot_general(
        p.astype(jnp.bfloat16), v_ref[...].astype(jnp.bfloat16),
        (((1,), (1,)), ((), ())),
        preferred_element_type=jnp.float32)
    acc_ref[...] = acc_ref[...] * alpha[:, 0:1] + pv

    @pl.when(i == n - 1)
    def _finish():
        o_ref[...] = acc_ref[...] / l_ref[...][:, 0:1]


def kernel(x, memory_keys, memory_values):
    kT = memory_keys.T
    vT = memory_values.T
    m_total = memory_keys.shape[0]
    grid = (m_total // _BLOCK,)
    return pl.pallas_call(
        _flash_kernel,
        grid=grid,
        in_specs=[
            pl.BlockSpec((_B, _D), lambda i: (0, 0)),
            pl.BlockSpec((_D, _BLOCK), lambda i: (0, i)),
            pl.BlockSpec((_D, _BLOCK), lambda i: (0, i)),
        ],
        out_specs=pl.BlockSpec((_B, _D), lambda i: (0, 0)),
        out_shape=jax.ShapeDtypeStruct((_B, _D), jnp.float32),
        scratch_shapes=[
            pltpu.VMEM((_B, 128), jnp.float32),
            pltpu.VMEM((_B, 128), jnp.float32),
            pltpu.VMEM((_B, _D), jnp.float32),
        ],
        compiler_params=pltpu.CompilerParams(
            dimension_semantics=("parallel",),
        ),
    )(x, kT, vT)
